# baseline (speedup 1.0000x reference)
import functools

import jax
import jax.numpy as jnp
from jax import lax
from jax.experimental import pallas as pl
from jax.experimental.pallas import tpu as pltpu
from jax.experimental.pallas import tpu_sc as plsc

_N = 100000
_E = 1600000
_D = 6.0

_LANES = 128
_NPAD = 100352
_ROWS_X = _NPAD // _LANES

_NC = 2
_NS = 16
_NW = _NC * _NS

_PAIRS = _E // _LANES
_BP = 32
_KB2 = 2 * _BP
_NBLK = _PAIRS // _BP
_TAILP = _PAIRS - _NBLK * _BP
_BASE = _NBLK // _NW
_EXTRA = _NBLK - _BASE * _NW

_SLICE = _NPAD // _NS


def _sc_spmm(x_pad, ei_rows, zeros):
    mesh = plsc.VectorSubcoreMesh(core_axis_name="c", subcore_axis_name="s")

    @functools.partial(
        pl.kernel,
        mesh=mesh,
        out_type=jax.ShapeDtypeStruct((_NC * _NPAD,), jnp.float32),
        compiler_params=pltpu.CompilerParams(needs_layout_passes=False,
                                             use_tc_tiling_on_sc=False),
        scratch_types=[
            [pltpu.VMEM((_BP, _LANES), jnp.int32)] * 2,
            [pltpu.VMEM((_BP, _LANES), jnp.int32)] * 2,
            [pltpu.VMEM((_BP, _LANES), jnp.float32)] * 2,
            pltpu.VMEM_SHARED((_NPAD,), jnp.float32),
            pltpu.VMEM_SHARED((_NPAD,), jnp.float32),
            [pltpu.SemaphoreType.DMA] * 2,
            [pltpu.SemaphoreType.DMA] * 2,
            [pltpu.SemaphoreType.DMA] * 2,
        ],
    )
    def k(x_hbm, ei_hbm, zeros_hbm, out_hbm, srcb, dstb, gathb, xs, acc,
          isem, gsem, ssem):
        cid = lax.axis_index("c")
        sid = lax.axis_index("s")

        pltpu.sync_copy(zeros_hbm.at[pl.ds(sid * _SLICE, _SLICE)],
                        acc.at[pl.ds(sid * _SLICE, _SLICE)])
        pltpu.sync_copy(x_hbm.at[pl.ds(sid * _SLICE, _SLICE)],
                        xs.at[pl.ds(sid * _SLICE, _SLICE)])
        plsc.subcore_barrier()

        w = cid * _NS + sid
        nb_w = _BASE + jnp.where(w < _EXTRA, 1, 0)
        start_w = _BASE * w + jnp.minimum(w, _EXTRA)

        def fire_idx(pairs, blk, u):
            rows = pl.ds(blk * _BP, pairs)
            return [pltpu.async_copy(ei_hbm.at[rows, 0],
                                     srcb[u].at[pl.ds(0, pairs)], isem[u]),
                    pltpu.async_copy(ei_hbm.at[rows, 1],
                                     dstb[u].at[pl.ds(0, pairs)], isem[u])]

        def fire_gath(u, pairs):
            return [pltpu.async_copy(xs.at[srcb[u].at[j]], gathb[u].at[j],
                                     gsem[u])
                    for j in range(pairs)]

        def fire_scat(u, pairs):
            return [pltpu.async_copy(gathb[u].at[j], acc.at[dstb[u].at[j]],
                                     ssem[u], add=True)
                    for j in range(pairs)]

        def drain(ds):
            for d in ds:
                d.wait()

        def slot(g, _):
            b0 = start_w + 2 * g
            i0 = fire_idx(_BP, b0, 0)
            i1 = fire_idx(_BP, b0 + 1, 1)
            drain(i0)
            g0 = fire_gath(0, _BP)
            drain(i1)
            drain(g0)
            s0 = fire_scat(0, _BP)
            g1 = fire_gath(1, _BP)
            drain(g1)
            s1 = fire_scat(1, _BP)
            drain(s0)
            drain(s1)
            return ()

        lax.fori_loop(0, nb_w // 2, slot, (), unroll=False)

        @pl.when(nb_w % 2 == 1)
        def _():
            drain(fire_idx(_BP, start_w + nb_w - 1, 0))
            drain(fire_gath(0, _BP))
            drain(fire_scat(0, _BP))

        @pl.when(w == _NW - 1)
        def _():
            drain(fire_idx(_TAILP, _NBLK, 1))
            drain(fire_gath(1, _TAILP))
            drain(fire_scat(1, _TAILP))

        plsc.subcore_barrier()

        pltpu.sync_copy(acc.at[pl.ds(sid * _SLICE, _SLICE)],
                        out_hbm.at[pl.ds(cid * _NPAD + sid * _SLICE, _SLICE)])

    return k(x_pad, ei_rows, zeros)


def _tc_combine(x2d, partials):
    def body(x_ref, p_ref, o_ref):
        xx = x_ref[...]
        ax = p_ref[0] + p_ref[1]
        o_ref[...] = (-_D) * xx + (1.0 - xx) * ax

    return pl.pallas_call(
        body,
        out_shape=jax.ShapeDtypeStruct((_ROWS_X, _LANES), jnp.float32),
    )(x2d, partials)


def kernel(t, x, edge_index):
    del t
    x_flat = x[:, 0]
    x_pad = jnp.pad(x_flat, (0, _NPAD - _N))
    ei_rows = (edge_index.reshape(2, _PAIRS, _LANES)
               .transpose(1, 0, 2))
    zeros = jnp.zeros((_NPAD,), jnp.float32)

    partials = _sc_spmm(x_pad, ei_rows, zeros)
    out2d = _tc_combine(x_pad.reshape(_ROWS_X, _LANES),
                        partials.reshape(_NC, _ROWS_X, _LANES))
    return out2d.reshape(-1)[:_N].reshape(_N, 1)


# speedup vs baseline: 1.0577x; 1.0577x over previous
import functools

import jax
import jax.numpy as jnp
from jax import lax
from jax.experimental import pallas as pl
from jax.experimental.pallas import tpu as pltpu
from jax.experimental.pallas import tpu_sc as plsc

_N = 100000
_E = 1600000
_D = 6.0

_LANES = 128
_NPAD = 100352
_ROWS_X = _NPAD // _LANES

_NC = 2
_NS = 16
_NW = _NC * _NS

_PAIRS = _E // _LANES
_BP = 16
_NBLK = _PAIRS // _BP
_TAILP = _PAIRS - _NBLK * _BP
_BASE = _NBLK // _NW
_EXTRA = _NBLK - _BASE * _NW

_SLICE = _NPAD // _NS


def _sc_spmm(x_pad, ei_rows, zeros):
    mesh = plsc.VectorSubcoreMesh(core_axis_name="c", subcore_axis_name="s")

    @functools.partial(
        pl.kernel,
        mesh=mesh,
        out_type=jax.ShapeDtypeStruct((_NC * _NPAD,), jnp.float32),
        compiler_params=pltpu.CompilerParams(needs_layout_passes=False,
                                             use_tc_tiling_on_sc=False),
        scratch_types=[
            [pltpu.VMEM((_BP, _LANES), jnp.int32)] * 2,
            [pltpu.VMEM((_BP, _LANES), jnp.int32)] * 2,
            [pltpu.VMEM((_BP, _LANES), jnp.float32)] * 2,
            pltpu.VMEM((_NPAD,), jnp.float32),
            pltpu.VMEM_SHARED((_NPAD,), jnp.float32),
            [pltpu.SemaphoreType.DMA] * 2,
            [pltpu.SemaphoreType.DMA] * 2,
        ],
    )
    def k(x_hbm, ei_hbm, zeros_hbm, out_hbm, srcb, dstb, gathb, xv, acc,
          isem, ssem):
        cid = lax.axis_index("c")
        sid = lax.axis_index("s")

        pltpu.sync_copy(zeros_hbm.at[pl.ds(sid * _SLICE, _SLICE)],
                        acc.at[pl.ds(sid * _SLICE, _SLICE)])
        pltpu.sync_copy(x_hbm, xv)
        plsc.subcore_barrier()

        w = cid * _NS + sid
        nb_w = _BASE + jnp.where(w < _EXTRA, 1, 0)
        start_w = _BASE * w + jnp.minimum(w, _EXTRA)

        def fire_idx(pairs, blk, u):
            rows = pl.ds(blk * _BP, pairs)
            return [pltpu.async_copy(ei_hbm.at[rows, 0],
                                     srcb[u].at[pl.ds(0, pairs)], isem[u]),
                    pltpu.async_copy(ei_hbm.at[rows, 1],
                                     dstb[u].at[pl.ds(0, pairs)], isem[u])]

        def gath_block(u, pairs):
            for j in range(pairs):
                for g in range(_LANES // 16):
                    idx16 = srcb[u][j, pl.ds(g * 16, 16)]
                    gathb[u][j, pl.ds(g * 16, 16)] = plsc.load_gather(
                        xv, [idx16])

        def fire_scat(u, pairs):
            return [pltpu.async_copy(gathb[u].at[j], acc.at[dstb[u].at[j]],
                                     ssem[u], add=True)
                    for j in range(pairs)]

        def drain(ds):
            for d in ds:
                d.wait()

        def slot(g, _):
            b0 = start_w + 2 * g
            i0 = fire_idx(_BP, b0, 0)
            i1 = fire_idx(_BP, b0 + 1, 1)
            drain(i0)
            gath_block(0, _BP)
            drain(i1)
            s0 = fire_scat(0, _BP)
            gath_block(1, _BP)
            s1 = fire_scat(1, _BP)
            drain(s0)
            drain(s1)
            return ()

        lax.fori_loop(0, nb_w // 2, slot, (), unroll=False)

        @pl.when(nb_w % 2 == 1)
        def _():
            drain(fire_idx(_BP, start_w + nb_w - 1, 0))
            gath_block(0, _BP)
            drain(fire_scat(0, _BP))

        @pl.when(w == _NW - 1)
        def _():
            drain(fire_idx(_TAILP, _NBLK, 1))
            gath_block(1, _TAILP)
            drain(fire_scat(1, _TAILP))

        plsc.subcore_barrier()

        pltpu.sync_copy(acc.at[pl.ds(sid * _SLICE, _SLICE)],
                        out_hbm.at[pl.ds(cid * _NPAD + sid * _SLICE, _SLICE)])

    return k(x_pad, ei_rows, zeros)


def _tc_combine(x2d, partials):
    def body(x_ref, p_ref, o_ref):
        xx = x_ref[...]
        ax = p_ref[0] + p_ref[1]
        o_ref[...] = (-_D) * xx + (1.0 - xx) * ax

    return pl.pallas_call(
        body,
        out_shape=jax.ShapeDtypeStruct((_ROWS_X, _LANES), jnp.float32),
    )(x2d, partials)


def kernel(t, x, edge_index):
    del t
    x_flat = x[:, 0]
    x_pad = jnp.pad(x_flat, (0, _NPAD - _N))
    ei_rows = (edge_index.reshape(2, _PAIRS, _LANES)
               .transpose(1, 0, 2))
    zeros = jnp.zeros((_NPAD,), jnp.float32)

    partials = _sc_spmm(x_pad, ei_rows, zeros)
    out2d = _tc_combine(x_pad.reshape(_ROWS_X, _LANES),
                        partials.reshape(_NC, _ROWS_X, _LANES))
    return out2d.reshape(-1)[:_N].reshape(_N, 1)


# speedup vs baseline: 1.0601x; 1.0022x over previous
import functools

import jax
import jax.numpy as jnp
from jax import lax
from jax.experimental import pallas as pl
from jax.experimental.pallas import tpu as pltpu
from jax.experimental.pallas import tpu_sc as plsc

_N = 100000
_E = 1600000
_D = 6.0

_LANES = 128
_NPAD = 100352
_ROWS_X = _NPAD // _LANES

_NC = 2
_NS = 16
_NW = _NC * _NS

_PAIRS = _E // _LANES
_BP = 16
_NBLK = _PAIRS // _BP
_TAILP = _PAIRS - _NBLK * _BP
_BASE = _NBLK // _NW
_EXTRA = _NBLK - _BASE * _NW

_SLICE = _NPAD // _NS


def _sc_spmm(x_pad, ei_rows, zeros):
    mesh = plsc.VectorSubcoreMesh(core_axis_name="c", subcore_axis_name="s")

    @functools.partial(
        pl.kernel,
        mesh=mesh,
        out_type=jax.ShapeDtypeStruct((_NC * _NPAD,), jnp.float32),
        compiler_params=pltpu.CompilerParams(needs_layout_passes=False,
                                             use_tc_tiling_on_sc=False),
        scratch_types=[
            [pltpu.VMEM((_BP, _LANES), jnp.int32)] * 2,
            [pltpu.VMEM((_BP, _LANES), jnp.int32)] * 2,
            [pltpu.VMEM((1, _BP * _LANES), jnp.int32)] * 2,
            [pltpu.VMEM((1, _BP * _LANES), jnp.float32)] * 2,
            pltpu.VMEM((_NPAD,), jnp.float32),
            pltpu.VMEM_SHARED((_NPAD,), jnp.float32),
            [pltpu.SemaphoreType.DMA] * 2,
            [pltpu.SemaphoreType.DMA] * 2,
        ],
    )
    def k(x_hbm, ei_hbm, zeros_hbm, out_hbm, srcb, dstb, dst1, gathb, xv,
          acc, isem, ssem):
        cid = lax.axis_index("c")
        sid = lax.axis_index("s")

        pltpu.sync_copy(zeros_hbm.at[pl.ds(sid * _SLICE, _SLICE)],
                        acc.at[pl.ds(sid * _SLICE, _SLICE)])
        pltpu.sync_copy(x_hbm, xv)
        plsc.subcore_barrier()

        w = cid * _NS + sid
        nb_w = _BASE + jnp.where(w < _EXTRA, 1, 0)
        start_w = _BASE * w + jnp.minimum(w, _EXTRA)

        def fire_idx(pairs, blk, u):
            rows = pl.ds(blk * _BP, pairs)
            return [pltpu.async_copy(ei_hbm.at[rows, 0],
                                     srcb[u].at[pl.ds(0, pairs)], isem[u]),
                    pltpu.async_copy(ei_hbm.at[rows, 1],
                                     dstb[u].at[pl.ds(0, pairs)], isem[u])]

        def gath_block(u, pairs):
            for j in range(pairs):
                for g in range(_LANES // 16):
                    sl = pl.ds(g * 16, 16)
                    fl = pl.ds(j * _LANES + g * 16, 16)
                    gathb[u][0, fl] = plsc.load_gather(xv, [srcb[u][j, sl]])
                    dst1[u][0, fl] = dstb[u][j, sl]

        def fire_scat(u, pairs):
            n = pairs * _LANES
            return [pltpu.async_copy(gathb[u].at[0, pl.ds(0, n)],
                                     acc.at[dst1[u].at[0, pl.ds(0, n)]],
                                     ssem[u], add=True)]

        def drain(ds):
            for d in ds:
                d.wait()

        def slot(g, _):
            b0 = start_w + 2 * g
            i0 = fire_idx(_BP, b0, 0)
            i1 = fire_idx(_BP, b0 + 1, 1)
            drain(i0)
            gath_block(0, _BP)
            drain(i1)
            s0 = fire_scat(0, _BP)
            gath_block(1, _BP)
            s1 = fire_scat(1, _BP)
            drain(s0)
            drain(s1)
            return ()

        lax.fori_loop(0, nb_w // 2, slot, (), unroll=False)

        @pl.when(nb_w % 2 == 1)
        def _():
            drain(fire_idx(_BP, start_w + nb_w - 1, 0))
            gath_block(0, _BP)
            drain(fire_scat(0, _BP))

        @pl.when(w == _NW - 1)
        def _():
            drain(fire_idx(_TAILP, _NBLK, 1))
            gath_block(1, _TAILP)
            drain(fire_scat(1, _TAILP))

        plsc.subcore_barrier()

        pltpu.sync_copy(acc.at[pl.ds(sid * _SLICE, _SLICE)],
                        out_hbm.at[pl.ds(cid * _NPAD + sid * _SLICE, _SLICE)])

    return k(x_pad, ei_rows, zeros)


def _tc_combine(x2d, partials):
    def body(x_ref, p_ref, o_ref):
        xx = x_ref[...]
        ax = p_ref[0] + p_ref[1]
        o_ref[...] = (-_D) * xx + (1.0 - xx) * ax

    return pl.pallas_call(
        body,
        out_shape=jax.ShapeDtypeStruct((_ROWS_X, _LANES), jnp.float32),
    )(x2d, partials)


def kernel(t, x, edge_index):
    del t
    x_flat = x[:, 0]
    x_pad = jnp.pad(x_flat, (0, _NPAD - _N))
    ei_rows = (edge_index.reshape(2, _PAIRS, _LANES)
               .transpose(1, 0, 2))
    zeros = jnp.zeros((_NPAD,), jnp.float32)

    partials = _sc_spmm(x_pad, ei_rows, zeros)
    out2d = _tc_combine(x_pad.reshape(_ROWS_X, _LANES),
                        partials.reshape(_NC, _ROWS_X, _LANES))
    return out2d.reshape(-1)[:_N].reshape(_N, 1)
